# Initial kernel scaffold; baseline (speedup 1.0000x reference)
import jax
import jax.numpy as jnp
from jax.experimental import pallas as pl


def kernel(queries, keys):
    raise NotImplementedError("write your pallas kernel here")


# speedup vs baseline: 1.6647x; 1.6647x over previous
import functools

import jax
import jax.numpy as jnp
from jax import lax
from jax.experimental import pallas as pl

TOPK = 16
KB = 2048
INF = float("inf")
IMAX = 2**31 - 1


def _topk_body(n_valid, q_ref, kt_ref, q2_ref, k2_ref, vals_ref, idx_ref):
    kb = pl.program_id(0)

    @pl.when(kb == 0)
    def _init():
        vals_ref[...] = jnp.full(vals_ref.shape, INF, jnp.float32)
        idx_ref[...] = jnp.zeros(idx_ref.shape, jnp.int32)

    q = q_ref[...]
    kt = kt_ref[...]
    q2 = q2_ref[...]
    k2 = k2_ref[...]
    qk = jnp.dot(q, kt, preferred_element_type=jnp.float32)
    d2 = q2 + k2 - 2.0 * qk
    d2 = jnp.maximum(d2, 0.0)

    gidx = lax.broadcasted_iota(jnp.int32, d2.shape, 1) + kb * KB
    d2 = jnp.where(gidx < n_valid, d2, INF)

    bv, bi = [], []
    vals, idx = d2, gidx
    for _ in range(TOPK):
        m = jnp.min(vals, axis=1, keepdims=True)
        sel = jnp.min(jnp.where(vals == m, idx, IMAX), axis=1, keepdims=True)
        bv.append(m)
        bi.append(sel)
        vals = jnp.where(idx == sel, INF, vals)

    cv = jnp.concatenate([vals_ref[...]] + bv, axis=1)
    ci = jnp.concatenate([idx_ref[...]] + bi, axis=1)
    nv, ni = [], []
    for _ in range(TOPK):
        m = jnp.min(cv, axis=1, keepdims=True)
        sel = jnp.min(jnp.where(cv == m, ci, IMAX), axis=1, keepdims=True)
        nv.append(m)
        ni.append(sel)
        cv = jnp.where(ci == sel, INF, cv)
    vals_ref[...] = jnp.concatenate(nv, axis=1)
    idx_ref[...] = jnp.concatenate(ni, axis=1)


def kernel(queries, keys):
    Q, D = queries.shape
    K = keys.shape[0]
    nkb = pl.cdiv(K, KB)
    kpad = nkb * KB

    q2 = jnp.sum(queries * queries, axis=1, keepdims=True)
    k2 = jnp.sum(keys * keys, axis=1)[None, :]
    keys_t = jnp.pad(keys.T, ((0, 0), (0, kpad - K)))
    k2p = jnp.pad(k2, ((0, 0), (0, kpad - K)))

    vals, idx = pl.pallas_call(
        functools.partial(_topk_body, K),
        grid=(nkb,),
        in_specs=[
            pl.BlockSpec((Q, D), lambda k: (0, 0)),
            pl.BlockSpec((D, KB), lambda k: (0, k)),
            pl.BlockSpec((Q, 1), lambda k: (0, 0)),
            pl.BlockSpec((1, KB), lambda k: (0, k)),
        ],
        out_specs=[
            pl.BlockSpec((Q, TOPK), lambda k: (0, 0)),
            pl.BlockSpec((Q, TOPK), lambda k: (0, 0)),
        ],
        out_shape=[
            jax.ShapeDtypeStruct((Q, TOPK), jnp.float32),
            jax.ShapeDtypeStruct((Q, TOPK), jnp.int32),
        ],
    )(queries, keys_t, q2, k2p)
    return (vals, idx)
